# baseline (speedup 1.0000x reference)
import functools

import jax
import jax.numpy as jnp
from jax import lax
from jax.experimental import pallas as pl
from jax.experimental.pallas import tpu as pltpu
from jax.experimental.pallas import tpu_sc as plsc

S = 128
V = S * S * S
NC, NS = 2, 16
NW = NC * NS
PW = V // NW

CB = 4096
NPAIR_B = PW // CB // 2
GB = CB // 16
RD2 = CB + 152

CW = 2048
NPAIR_W = PW // CW // 2
GW = CW // 16

_WOFFS = (0, 1, S, S + 1)

_params = pltpu.CompilerParams(needs_layout_passes=False,
                               use_tc_tiling_on_sc=False)
_mesh = lambda: plsc.VectorSubcoreMesh(core_axis_name="c",
                                       subcore_axis_name="s",
                                       num_cores=NC, num_subcores=NS)


def _pack(den, msk):
    d = plsc.bitcast(den, jnp.int32)
    return plsc.bitcast((d & -2) | msk.astype(jnp.int32), jnp.float32)


def _build_body(den_hbm, msk_hbm, tab_hbm, *sc):
    sets = (sc[:5], sc[5:10])
    semr = (sc[10], sc[11])
    semw = (sc[12], sc[13])
    wid = lax.axis_index("s") * NC + lax.axis_index("c")
    base_pt = wid * PW
    iota = lax.iota(jnp.int32, 16)
    ccs = [jnp.full((16,), c, jnp.int32) for c in range(8)]

    def read_descs(b, st, sem):
        rb0 = pl.multiple_of(jnp.minimum(b, V - RD2), 8)
        rb1 = pl.multiple_of(jnp.minimum(b + S * S, V - RD2), 8)
        cps = [pltpu.make_async_copy(den_hbm.at[pl.ds(rb0, RD2)], st[0], sem),
               pltpu.make_async_copy(den_hbm.at[pl.ds(rb1, RD2)], st[1], sem),
               pltpu.make_async_copy(msk_hbm.at[pl.ds(rb0, RD2)], st[2], sem),
               pltpu.make_async_copy(msk_hbm.at[pl.ds(rb1, RD2)], st[3], sem)]
        return cps, b - rb0, b + S * S - rb1

    def fire_reads(b, st, sem):
        cps, d0, d1 = read_descs(b, st, sem)
        for cp in cps:
            cp.start()
        return cps, d0, d1

    def assemble(b, st, d0, d1):
        sden0, sden1, smsk0, smsk1, rows = st

        def fast_g(g):
            pid = g * 16 + iota
            l0 = d0 + g * 16
            l1 = d1 + g * 16
            for c, off in enumerate(_WOFFS):
                plsc.store_scatter(rows, [pid, ccs[c]],
                                   _pack(sden0[pl.ds(l0 + off, 16)],
                                         smsk0[pl.ds(l0 + off, 16)]))
                plsc.store_scatter(rows, [pid, ccs[c + 4]],
                                   _pack(sden1[pl.ds(l1 + off, 16)],
                                         smsk1[pl.ds(l1 + off, 16)]))

        def fast(g, carry):
            fast_g(g)
            return carry

        def slow(g, carry):
            pid = g * 16 + iota
            l0 = d0 + g * 16
            l1 = d1 + g * 16
            for c, off in enumerate(_WOFFS):
                b0 = jnp.minimum(l0 + off + iota, RD2 - 1)
                b1 = jnp.minimum(l1 + off + iota, RD2 - 1)
                plsc.store_scatter(rows, [pid, ccs[c]],
                                   _pack(plsc.load_gather(sden0, [b0]),
                                         plsc.load_gather(smsk0, [b0])))
                plsc.store_scatter(rows, [pid, ccs[c + 4]],
                                   _pack(plsc.load_gather(sden1, [b1]),
                                         plsc.load_gather(smsk1, [b1])))
            return carry

        all_safe = b <= V - CB - 16560
        g_safe = jnp.clip((V - 16560 - b) // 16, 0, GB)

        @pl.when(all_safe)
        def _():
            plsc.parallel_loop(0, GB, unroll=8)(fast_g)

        @pl.when(jnp.logical_not(all_safe))
        def _():
            lax.fori_loop(0, g_safe, fast, 0)
            lax.fori_loop(g_safe, GB, slow, 0)

    def pair_body(ci2, carry):
        b0 = base_pt + (2 * ci2) * CB
        b1 = b0 + CB
        r0, d00, d01 = read_descs(b0, sets[0], semr[0])
        for cp in r0:
            cp.wait()
        assemble(b0, sets[0], d00, d01)
        w0 = pltpu.async_copy(sets[0][4], tab_hbm.at[pl.ds(b0, CB)], semw[0])

        @pl.when(ci2 < NPAIR_B - 1)
        def _():
            fire_reads(b0 + 2 * CB, sets[0], semr[0])

        r1, d10, d11 = read_descs(b1, sets[1], semr[1])
        for cp in r1:
            cp.wait()
        assemble(b1, sets[1], d10, d11)
        w1 = pltpu.async_copy(sets[1][4], tab_hbm.at[pl.ds(b1, CB)], semw[1])

        @pl.when(ci2 < NPAIR_B - 1)
        def _():
            fire_reads(b1 + 2 * CB, sets[1], semr[1])

        w0.wait()
        w1.wait()
        return carry

    fire_reads(base_pt, sets[0], semr[0])
    fire_reads(base_pt + CB, sets[1], semr[1])
    lax.fori_loop(0, NPAIR_B, pair_body, 0)


def _unnormalize(c):
    x = jnp.minimum(jnp.maximum(c - 0.5, 0.0), 127.0)
    xi = x.astype(jnp.int32)
    tx = x - xi.astype(jnp.float32)
    return xi, tx


def _round_bit(xi, tx):
    up = tx > 0.5
    tie = tx == 0.5
    odd = (xi & 1) == 1
    return jnp.where(up | (tie & odd), 1, 0)


def _warp_body(coords_hbm, tab_hbm, outd_hbm, outm_hbm, disp_hbm, *sc):
    sets = (sc[:14], sc[14:28])
    semr = (sc[28], sc[29])
    semg = (sc[30], sc[31])
    semw = (sc[32], sc[33])
    wid = lax.axis_index("s") * NC + lax.axis_index("c")
    base_pt = wid * PW
    iota = lax.iota(jnp.int32, 16)
    ccs = [jnp.full((16,), c, jnp.int32) for c in range(8)]

    def chunk_addr(cb):
        d = cb >> 14
        hw = cb & (S * S - 1)
        pbase = pl.multiple_of(d * 3 * (S * S) + hw, 8)
        return d, hw, pbase

    def read_descs(pbase, st, sem):
        return [pltpu.make_async_copy(coords_hbm.at[pl.ds(pbase, CW)],
                                      st[0], sem),
                pltpu.make_async_copy(coords_hbm.at[pl.ds(pbase + S * S, CW)],
                                      st[1], sem),
                pltpu.make_async_copy(
                    coords_hbm.at[pl.ds(pbase + 2 * S * S, CW)], st[2], sem)]

    def fire_reads(pbase, st, sem):
        cps = read_descs(pbase, st, sem)
        for cp in cps:
            cp.start()
        return cps

    def phase_a(st, d, hw):
        (cx_v, cy_v, cz_v, idx_v, msel_v, tx_v, ty_v, tz_v,
         rows_v, outd_v, outm_v, dx_v, dy_v, dz_v) = st
        d_f = d.astype(jnp.float32)

        def group_a(g):
            sl = pl.ds(g * 16, 16)
            cx = cx_v[sl]
            cy = cy_v[sl]
            cz = cz_v[sl]
            x0, tx = _unnormalize(cx)
            y0, ty = _unnormalize(cy)
            z0, tz = _unnormalize(cz)
            idx_v[sl] = (z0 << 14) + (y0 << 7) + x0
            msel_v[sl] = ((_round_bit(z0, tz) << 2)
                          + (_round_bit(y0, ty) << 1) + _round_bit(x0, tx))
            tx_v[sl] = tx
            ty_v[sl] = ty
            tz_v[sl] = tz
            hwl = hw + g * 16 + iota
            ph = (hwl >> 7).astype(jnp.float32)
            pw = (hwl & (S - 1)).astype(jnp.float32)
            dx_v[sl] = cx - d_f
            dy_v[sl] = cy - ph
            dz_v[sl] = cz - pw

        plsc.parallel_loop(0, GW, unroll=8)(group_a)

    def phase_b(st):
        (cx_v, cy_v, cz_v, idx_v, msel_v, tx_v, ty_v, tz_v,
         rows_v, outd_v, outm_v, dx_v, dy_v, dz_v) = st

        def group_b(g):
            pid = iota + g * 16
            v = [plsc.load_gather(rows_v, [pid, ccs[c]]) for c in range(8)]
            sl = pl.ds(g * 16, 16)
            tx = tx_v[sl]
            ty = ty_v[sl]
            tz = tz_v[sl]
            l00 = v[0] + tx * (v[1] - v[0])
            l01 = v[2] + tx * (v[3] - v[2])
            l10 = v[4] + tx * (v[5] - v[4])
            l11 = v[6] + tx * (v[7] - v[6])
            m0 = l00 + ty * (l01 - l00)
            m1 = l10 + ty * (l11 - l10)
            outd_v[sl] = m0 + tz * (m1 - m0)
            mw = plsc.load_gather(rows_v, [pid, msel_v[sl]])
            outm_v[sl] = (plsc.bitcast(mw, jnp.int32) & 1).astype(jnp.float32)

        plsc.parallel_loop(0, GW, unroll=8)(group_b)

    def fire_writes(cb, pbase, st, sem):
        return [pltpu.async_copy(st[9], outd_hbm.at[pl.ds(cb, CW)], sem),
                pltpu.async_copy(st[10], outm_hbm.at[pl.ds(cb, CW)], sem),
                pltpu.async_copy(st[11], disp_hbm.at[pl.ds(pbase, CW)], sem),
                pltpu.async_copy(st[12],
                                 disp_hbm.at[pl.ds(pbase + S * S, CW)], sem),
                pltpu.async_copy(st[13],
                                 disp_hbm.at[pl.ds(pbase + 2 * S * S, CW)],
                                 sem)]

    def pair_body(ci2, carry):
        cb0 = base_pt + (2 * ci2) * CW
        cb1 = cb0 + CW
        d0, hw0, pb0 = chunk_addr(cb0)
        d1, hw1, pb1 = chunk_addr(cb1)
        for cp in read_descs(pb0, sets[0], semr[0]):
            cp.wait()
        phase_a(sets[0], d0, hw0)
        g0 = pltpu.async_copy(tab_hbm.at[sets[0][3]], sets[0][8], semg[0])

        @pl.when(ci2 < NPAIR_W - 1)
        def _():
            _, _, pbn = chunk_addr(cb0 + 2 * CW)
            fire_reads(pbn, sets[0], semr[0])

        for cp in read_descs(pb1, sets[1], semr[1]):
            cp.wait()
        phase_a(sets[1], d1, hw1)
        g1 = pltpu.async_copy(tab_hbm.at[sets[1][3]], sets[1][8], semg[1])

        @pl.when(ci2 < NPAIR_W - 1)
        def _():
            _, _, pbn = chunk_addr(cb1 + 2 * CW)
            fire_reads(pbn, sets[1], semr[1])

        g0.wait()
        phase_b(sets[0])
        w0 = fire_writes(cb0, pb0, sets[0], semw[0])
        g1.wait()
        phase_b(sets[1])
        w1 = fire_writes(cb1, pb1, sets[1], semw[1])
        for cp in w0 + w1:
            cp.wait()
        return carry

    _, _, pbp0 = chunk_addr(base_pt)
    _, _, pbp1 = chunk_addr(base_pt + CW)
    fire_reads(pbp0, sets[0], semr[0])
    fire_reads(pbp1, sets[1], semr[1])
    lax.fori_loop(0, NPAIR_W, pair_body, 0)


def _warp_set_types():
    return [
        pltpu.VMEM((CW,), jnp.float32),
        pltpu.VMEM((CW,), jnp.float32),
        pltpu.VMEM((CW,), jnp.float32),
        pltpu.VMEM((CW,), jnp.int32),
        pltpu.VMEM((CW,), jnp.int32),
        pltpu.VMEM((CW,), jnp.float32),
        pltpu.VMEM((CW,), jnp.float32),
        pltpu.VMEM((CW,), jnp.float32),
        pltpu.VMEM((CW, 8), jnp.float32),
        pltpu.VMEM((CW,), jnp.float32),
        pltpu.VMEM((CW,), jnp.float32),
        pltpu.VMEM((CW,), jnp.float32),
        pltpu.VMEM((CW,), jnp.float32),
        pltpu.VMEM((CW,), jnp.float32),
    ]


@functools.cache
def _get_build():
    bufset = [
        pltpu.VMEM((RD2,), jnp.float32),
        pltpu.VMEM((RD2,), jnp.float32),
        pltpu.VMEM((RD2,), jnp.float32),
        pltpu.VMEM((RD2,), jnp.float32),
        pltpu.VMEM((CB, 8), jnp.float32),
    ]
    return functools.partial(
        pl.kernel,
        out_type=jax.ShapeDtypeStruct((V, 8), jnp.float32),
        mesh=_mesh(),
        compiler_params=_params,
        scratch_types=(bufset + bufset
                       + [pltpu.SemaphoreType.DMA] * 4),
    )(_build_body)


@functools.cache
def _get_warp():
    return functools.partial(
        pl.kernel,
        out_type=(
            jax.ShapeDtypeStruct((V,), jnp.float32),
            jax.ShapeDtypeStruct((V,), jnp.float32),
            jax.ShapeDtypeStruct((V * 3,), jnp.float32),
        ),
        mesh=_mesh(),
        compiler_params=_params,
        scratch_types=(_warp_set_types() + _warp_set_types()
                       + [pltpu.SemaphoreType.DMA] * 6),
    )(_warp_body)


def kernel(density, mask, warped_coords):
    den = density.reshape(V)
    msk = mask.reshape(V)
    coords = warped_coords.transpose(0, 1, 4, 2, 3).reshape(V * 3)
    tab = _get_build()(den, msk)
    outd, outm, disp = _get_warp()(coords, tab)
    disp = disp.reshape(S, 3, S, S).transpose(0, 2, 3, 1)
    return (outd.reshape(S, S, S), outm.reshape(S, S, S), disp)


# speedup vs baseline: 1.5082x; 1.5082x over previous
import functools

import jax
import jax.numpy as jnp
from jax import lax
from jax.experimental import pallas as pl
from jax.experimental.pallas import tpu as pltpu
from jax.experimental.pallas import tpu_sc as plsc

S = 128
V = S * S * S
NC, NS = 2, 16
NW = NC * NS
PW = V // NW

CB = 4096
NPAIR_B = PW // CB // 2
GB = CB // 16
RD2 = CB + 152

CW = 2048
NPAIR_W = PW // CW // 2
GW = CW // 16

_WOFFS = (0, 1, S, S + 1)

_params = pltpu.CompilerParams(needs_layout_passes=False,
                               use_tc_tiling_on_sc=False)
_mesh = lambda: plsc.VectorSubcoreMesh(core_axis_name="c",
                                       subcore_axis_name="s",
                                       num_cores=NC, num_subcores=NS)


def _pack(den, msk):
    d = plsc.bitcast(den, jnp.int32)
    return plsc.bitcast((d & -2) | msk.astype(jnp.int32), jnp.float32)


def _build_body(den_hbm, msk_hbm, tab_hbm, *sc):
    sets = (sc[:5], sc[5:10])
    semr = (sc[10], sc[11])
    semw = (sc[12], sc[13])
    wid = lax.axis_index("s") * NC + lax.axis_index("c")
    base_pt = wid * PW
    iota = lax.iota(jnp.int32, 16)
    ccs = [jnp.full((16,), c, jnp.int32) for c in range(8)]

    def read_descs(b, st, sem):
        rb0 = pl.multiple_of(jnp.minimum(b, V - RD2), 8)
        rb1 = pl.multiple_of(jnp.minimum(b + S * S, V - RD2), 8)
        cps = [pltpu.make_async_copy(den_hbm.at[pl.ds(rb0, RD2)], st[0], sem),
               pltpu.make_async_copy(den_hbm.at[pl.ds(rb1, RD2)], st[1], sem),
               pltpu.make_async_copy(msk_hbm.at[pl.ds(rb0, RD2)], st[2], sem),
               pltpu.make_async_copy(msk_hbm.at[pl.ds(rb1, RD2)], st[3], sem)]
        return cps, b - rb0, b + S * S - rb1

    def fire_reads(b, st, sem):
        cps, d0, d1 = read_descs(b, st, sem)
        for cp in cps:
            cp.start()
        return cps, d0, d1

    def assemble(b, st, d0, d1):
        sden0, sden1, smsk0, smsk1, rows = st

        def fast_g(g):
            pid = g * 16 + iota
            l0 = d0 + g * 16
            l1 = d1 + g * 16
            for c, off in enumerate(_WOFFS):
                plsc.store_scatter(rows, [pid, ccs[c]],
                                   _pack(sden0[pl.ds(l0 + off, 16)],
                                         smsk0[pl.ds(l0 + off, 16)]))
                plsc.store_scatter(rows, [pid, ccs[c + 4]],
                                   _pack(sden1[pl.ds(l1 + off, 16)],
                                         smsk1[pl.ds(l1 + off, 16)]))

        def fast(g, carry):
            fast_g(g)
            return carry

        def slow(g, carry):
            pid = g * 16 + iota
            l0 = d0 + g * 16
            l1 = d1 + g * 16
            for c, off in enumerate(_WOFFS):
                b0 = jnp.minimum(l0 + off + iota, RD2 - 1)
                b1 = jnp.minimum(l1 + off + iota, RD2 - 1)
                plsc.store_scatter(rows, [pid, ccs[c]],
                                   _pack(plsc.load_gather(sden0, [b0]),
                                         plsc.load_gather(smsk0, [b0])))
                plsc.store_scatter(rows, [pid, ccs[c + 4]],
                                   _pack(plsc.load_gather(sden1, [b1]),
                                         plsc.load_gather(smsk1, [b1])))
            return carry

        all_safe = b <= V - CB - 16560
        g_safe = jnp.clip((V - 16560 - b) // 16, 0, GB)

        @pl.when(all_safe)
        def _():
            plsc.parallel_loop(0, GB, unroll=4)(fast_g)

        @pl.when(jnp.logical_not(all_safe))
        def _():
            lax.fori_loop(0, g_safe, fast, 0)
            lax.fori_loop(g_safe, GB, slow, 0)

    def pair_body(ci2, carry):
        b0 = base_pt + (2 * ci2) * CB
        b1 = b0 + CB
        r0, d00, d01 = read_descs(b0, sets[0], semr[0])
        for cp in r0:
            cp.wait()
        assemble(b0, sets[0], d00, d01)
        w0 = pltpu.async_copy(sets[0][4], tab_hbm.at[pl.ds(b0, CB)], semw[0])

        @pl.when(ci2 < NPAIR_B - 1)
        def _():
            fire_reads(b0 + 2 * CB, sets[0], semr[0])

        r1, d10, d11 = read_descs(b1, sets[1], semr[1])
        for cp in r1:
            cp.wait()
        assemble(b1, sets[1], d10, d11)
        w1 = pltpu.async_copy(sets[1][4], tab_hbm.at[pl.ds(b1, CB)], semw[1])

        @pl.when(ci2 < NPAIR_B - 1)
        def _():
            fire_reads(b1 + 2 * CB, sets[1], semr[1])

        w0.wait()
        w1.wait()
        return carry

    fire_reads(base_pt, sets[0], semr[0])
    fire_reads(base_pt + CB, sets[1], semr[1])
    lax.fori_loop(0, NPAIR_B, pair_body, 0)


def _unnormalize(c):
    x = jnp.minimum(jnp.maximum(c - 0.5, 0.0), 127.0)
    xi = x.astype(jnp.int32)
    tx = x - xi.astype(jnp.float32)
    return xi, tx


def _round_bit(xi, tx):
    up = tx > 0.5
    tie = tx == 0.5
    odd = (xi & 1) == 1
    return jnp.where(up | (tie & odd), 1, 0)


def _warp_body(coords_hbm, tab_hbm, outd_hbm, outm_hbm, disp_hbm, *sc):
    sets = (sc[:14], sc[14:28])
    semr = (sc[28], sc[29])
    semg = (sc[30], sc[31])
    semw = (sc[32], sc[33])
    wid = lax.axis_index("s") * NC + lax.axis_index("c")
    base_pt = wid * PW
    iota = lax.iota(jnp.int32, 16)
    ccs = [jnp.full((16,), c, jnp.int32) for c in range(8)]

    def chunk_addr(cb):
        d = cb >> 14
        hw = cb & (S * S - 1)
        pbase = pl.multiple_of(d * 3 * (S * S) + hw, 8)
        return d, hw, pbase

    def read_descs(pbase, st, sem):
        return [pltpu.make_async_copy(coords_hbm.at[pl.ds(pbase, CW)],
                                      st[0], sem),
                pltpu.make_async_copy(coords_hbm.at[pl.ds(pbase + S * S, CW)],
                                      st[1], sem),
                pltpu.make_async_copy(
                    coords_hbm.at[pl.ds(pbase + 2 * S * S, CW)], st[2], sem)]

    def fire_reads(pbase, st, sem):
        cps = read_descs(pbase, st, sem)
        for cp in cps:
            cp.start()
        return cps

    def phase_a(st, d, hw):
        (cx_v, cy_v, cz_v, idx_v, msel_v, tx_v, ty_v, tz_v,
         rows_v, outd_v, outm_v, dx_v, dy_v, dz_v) = st
        d_f = d.astype(jnp.float32)

        def group_a(g):
            sl = pl.ds(g * 16, 16)
            cx = cx_v[sl]
            cy = cy_v[sl]
            cz = cz_v[sl]
            x0, tx = _unnormalize(cx)
            y0, ty = _unnormalize(cy)
            z0, tz = _unnormalize(cz)
            idx_v[sl] = (z0 << 14) + (y0 << 7) + x0
            msel_v[sl] = ((_round_bit(z0, tz) << 2)
                          + (_round_bit(y0, ty) << 1) + _round_bit(x0, tx))
            tx_v[sl] = tx
            ty_v[sl] = ty
            tz_v[sl] = tz
            hwl = hw + g * 16 + iota
            ph = (hwl >> 7).astype(jnp.float32)
            pw = (hwl & (S - 1)).astype(jnp.float32)
            dx_v[sl] = cx - d_f
            dy_v[sl] = cy - ph
            dz_v[sl] = cz - pw

        plsc.parallel_loop(0, GW, unroll=4)(group_a)

    def phase_b(st):
        (cx_v, cy_v, cz_v, idx_v, msel_v, tx_v, ty_v, tz_v,
         rows_v, outd_v, outm_v, dx_v, dy_v, dz_v) = st

        def group_b(g):
            pid = iota + g * 16
            v = [plsc.load_gather(rows_v, [pid, ccs[c]]) for c in range(8)]
            sl = pl.ds(g * 16, 16)
            tx = tx_v[sl]
            ty = ty_v[sl]
            tz = tz_v[sl]
            l00 = v[0] + tx * (v[1] - v[0])
            l01 = v[2] + tx * (v[3] - v[2])
            l10 = v[4] + tx * (v[5] - v[4])
            l11 = v[6] + tx * (v[7] - v[6])
            m0 = l00 + ty * (l01 - l00)
            m1 = l10 + ty * (l11 - l10)
            outd_v[sl] = m0 + tz * (m1 - m0)
            mw = plsc.load_gather(rows_v, [pid, msel_v[sl]])
            outm_v[sl] = (plsc.bitcast(mw, jnp.int32) & 1).astype(jnp.float32)

        plsc.parallel_loop(0, GW, unroll=4)(group_b)

    def fire_writes(cb, pbase, st, sem):
        return [pltpu.async_copy(st[9], outd_hbm.at[pl.ds(cb, CW)], sem),
                pltpu.async_copy(st[10], outm_hbm.at[pl.ds(cb, CW)], sem),
                pltpu.async_copy(st[11], disp_hbm.at[pl.ds(pbase, CW)], sem),
                pltpu.async_copy(st[12],
                                 disp_hbm.at[pl.ds(pbase + S * S, CW)], sem),
                pltpu.async_copy(st[13],
                                 disp_hbm.at[pl.ds(pbase + 2 * S * S, CW)],
                                 sem)]

    def pair_body(ci2, carry):
        cb0 = base_pt + (2 * ci2) * CW
        cb1 = cb0 + CW
        d0, hw0, pb0 = chunk_addr(cb0)
        d1, hw1, pb1 = chunk_addr(cb1)
        for cp in read_descs(pb0, sets[0], semr[0]):
            cp.wait()
        phase_a(sets[0], d0, hw0)
        g0 = pltpu.async_copy(tab_hbm.at[sets[0][3]], sets[0][8], semg[0])

        @pl.when(ci2 < NPAIR_W - 1)
        def _():
            _, _, pbn = chunk_addr(cb0 + 2 * CW)
            fire_reads(pbn, sets[0], semr[0])

        for cp in read_descs(pb1, sets[1], semr[1]):
            cp.wait()
        phase_a(sets[1], d1, hw1)
        g1 = pltpu.async_copy(tab_hbm.at[sets[1][3]], sets[1][8], semg[1])

        @pl.when(ci2 < NPAIR_W - 1)
        def _():
            _, _, pbn = chunk_addr(cb1 + 2 * CW)
            fire_reads(pbn, sets[1], semr[1])

        g0.wait()
        phase_b(sets[0])
        w0 = fire_writes(cb0, pb0, sets[0], semw[0])
        g1.wait()
        phase_b(sets[1])
        w1 = fire_writes(cb1, pb1, sets[1], semw[1])
        for cp in w0 + w1:
            cp.wait()
        return carry

    _, _, pbp0 = chunk_addr(base_pt)
    _, _, pbp1 = chunk_addr(base_pt + CW)
    fire_reads(pbp0, sets[0], semr[0])
    fire_reads(pbp1, sets[1], semr[1])
    lax.fori_loop(0, NPAIR_W, pair_body, 0)


def _warp_set_types():
    return [
        pltpu.VMEM((CW,), jnp.float32),
        pltpu.VMEM((CW,), jnp.float32),
        pltpu.VMEM((CW,), jnp.float32),
        pltpu.VMEM((CW,), jnp.int32),
        pltpu.VMEM((CW,), jnp.int32),
        pltpu.VMEM((CW,), jnp.float32),
        pltpu.VMEM((CW,), jnp.float32),
        pltpu.VMEM((CW,), jnp.float32),
        pltpu.VMEM((CW, 8), jnp.float32),
        pltpu.VMEM((CW,), jnp.float32),
        pltpu.VMEM((CW,), jnp.float32),
        pltpu.VMEM((CW,), jnp.float32),
        pltpu.VMEM((CW,), jnp.float32),
        pltpu.VMEM((CW,), jnp.float32),
    ]


@functools.cache
def _get_build():
    bufset = [
        pltpu.VMEM((RD2,), jnp.float32),
        pltpu.VMEM((RD2,), jnp.float32),
        pltpu.VMEM((RD2,), jnp.float32),
        pltpu.VMEM((RD2,), jnp.float32),
        pltpu.VMEM((CB, 8), jnp.float32),
    ]
    return functools.partial(
        pl.kernel,
        out_type=jax.ShapeDtypeStruct((V, 8), jnp.float32),
        mesh=_mesh(),
        compiler_params=_params,
        scratch_types=(bufset + bufset
                       + [pltpu.SemaphoreType.DMA] * 4),
    )(_build_body)


@functools.cache
def _get_warp():
    return functools.partial(
        pl.kernel,
        out_type=(
            jax.ShapeDtypeStruct((V,), jnp.float32),
            jax.ShapeDtypeStruct((V,), jnp.float32),
            jax.ShapeDtypeStruct((V * 3,), jnp.float32),
        ),
        mesh=_mesh(),
        compiler_params=_params,
        scratch_types=(_warp_set_types() + _warp_set_types()
                       + [pltpu.SemaphoreType.DMA] * 6),
    )(_warp_body)


def kernel(density, mask, warped_coords):
    den = density.reshape(V)
    msk = mask.reshape(V)
    coords = warped_coords.transpose(0, 1, 4, 2, 3).reshape(V * 3)
    tab = _get_build()(den, msk)
    outd, outm, disp = _get_warp()(coords, tab)
    disp = disp.reshape(S, 3, S, S).transpose(0, 2, 3, 1)
    return (outd.reshape(S, S, S), outm.reshape(S, S, S), disp)


# speedup vs baseline: 1.7863x; 1.1844x over previous
import functools

import jax
import jax.numpy as jnp
from jax import lax
from jax.experimental import pallas as pl
from jax.experimental.pallas import tpu as pltpu
from jax.experimental.pallas import tpu_sc as plsc

S = 128
V = S * S * S
NC, NS = 2, 16
NW = NC * NS
PW = V // NW

CB = 4096
NPAIR_B = PW // CB // 2
GB = CB // 16
RD2 = CB + 152

CW = 2048
NPAIR_W = PW // CW // 2
GW = CW // 16

_WOFFS = (0, 1, S, S + 1)

_params = pltpu.CompilerParams(needs_layout_passes=False,
                               use_tc_tiling_on_sc=False)
_mesh = lambda: plsc.VectorSubcoreMesh(core_axis_name="c",
                                       subcore_axis_name="s",
                                       num_cores=NC, num_subcores=NS)


def _pack(den, msk):
    d = plsc.bitcast(den, jnp.int32)
    return plsc.bitcast((d & -2) | msk.astype(jnp.int32), jnp.float32)


def _build_body(den_hbm, msk_hbm, tab_hbm, *sc):
    sets = (sc[:5], sc[5:10])
    semr = (sc[10], sc[11])
    semw = (sc[12], sc[13])
    wid = lax.axis_index("s") * NC + lax.axis_index("c")
    base_pt = wid * PW
    iota = lax.iota(jnp.int32, 16)
    ccs = [jnp.full((16,), c, jnp.int32) for c in range(8)]

    def read_descs(b, st, sem):
        rb0 = pl.multiple_of(jnp.minimum(b, V - RD2), 8)
        rb1 = pl.multiple_of(jnp.minimum(b + S * S, V - RD2), 8)
        cps = [pltpu.make_async_copy(den_hbm.at[pl.ds(rb0, RD2)], st[0], sem),
               pltpu.make_async_copy(den_hbm.at[pl.ds(rb1, RD2)], st[1], sem),
               pltpu.make_async_copy(msk_hbm.at[pl.ds(rb0, RD2)], st[2], sem),
               pltpu.make_async_copy(msk_hbm.at[pl.ds(rb1, RD2)], st[3], sem)]
        return cps, b - rb0, b + S * S - rb1

    def fire_reads(b, st, sem):
        cps, d0, d1 = read_descs(b, st, sem)
        for cp in cps:
            cp.start()
        return cps, d0, d1

    def assemble(b, st, d0, d1):
        sden0, sden1, smsk0, smsk1, rows = st

        def fast_g(g):
            pid = g * 16 + iota
            l0 = d0 + g * 16
            l1 = d1 + g * 16
            for c, off in enumerate(_WOFFS):
                plsc.store_scatter(rows, [pid, ccs[c]],
                                   _pack(sden0[pl.ds(l0 + off, 16)],
                                         smsk0[pl.ds(l0 + off, 16)]))
                plsc.store_scatter(rows, [pid, ccs[c + 4]],
                                   _pack(sden1[pl.ds(l1 + off, 16)],
                                         smsk1[pl.ds(l1 + off, 16)]))

        def slow_g(g):
            pid = g * 16 + iota
            l0 = d0 + g * 16
            l1 = d1 + g * 16
            for c, off in enumerate(_WOFFS):
                b0 = jnp.minimum(l0 + off + iota, RD2 - 1)
                b1 = jnp.minimum(l1 + off + iota, RD2 - 1)
                plsc.store_scatter(rows, [pid, ccs[c]],
                                   _pack(plsc.load_gather(sden0, [b0]),
                                         plsc.load_gather(smsk0, [b0])))
                plsc.store_scatter(rows, [pid, ccs[c + 4]],
                                   _pack(plsc.load_gather(sden1, [b1]),
                                         plsc.load_gather(smsk1, [b1])))

        all_safe = b <= V - CB - 16560
        g_safe = jnp.clip((V - 16560 - b) // 16, 0, GB)

        @pl.when(all_safe)
        def _():
            plsc.parallel_loop(0, GB, unroll=4)(fast_g)

        @pl.when(jnp.logical_not(all_safe))
        def _():
            plsc.parallel_loop(0, g_safe)(fast_g)
            plsc.parallel_loop(g_safe, GB)(slow_g)

    def pair_body(ci2, carry):
        b0 = base_pt + (2 * ci2) * CB
        b1 = b0 + CB
        r0, d00, d01 = read_descs(b0, sets[0], semr[0])
        for cp in r0:
            cp.wait()
        assemble(b0, sets[0], d00, d01)
        w0 = pltpu.async_copy(sets[0][4], tab_hbm.at[pl.ds(b0, CB)], semw[0])

        @pl.when(ci2 < NPAIR_B - 1)
        def _():
            fire_reads(b0 + 2 * CB, sets[0], semr[0])

        r1, d10, d11 = read_descs(b1, sets[1], semr[1])
        for cp in r1:
            cp.wait()
        assemble(b1, sets[1], d10, d11)
        w1 = pltpu.async_copy(sets[1][4], tab_hbm.at[pl.ds(b1, CB)], semw[1])

        @pl.when(ci2 < NPAIR_B - 1)
        def _():
            fire_reads(b1 + 2 * CB, sets[1], semr[1])

        w0.wait()
        w1.wait()
        return carry

    fire_reads(base_pt, sets[0], semr[0])
    fire_reads(base_pt + CB, sets[1], semr[1])
    lax.fori_loop(0, NPAIR_B, pair_body, 0)


def _unnormalize(c):
    x = jnp.minimum(jnp.maximum(c - 0.5, 0.0), 127.0)
    xi = x.astype(jnp.int32)
    tx = x - xi.astype(jnp.float32)
    return xi, tx


def _round_bit(xi, tx):
    up = tx > 0.5
    tie = tx == 0.5
    odd = (xi & 1) == 1
    return jnp.where(up | (tie & odd), 1, 0)


def _warp_body(coords_hbm, tab_hbm, outd_hbm, outm_hbm, disp_hbm, *sc):
    sets = (sc[:14], sc[14:28])
    semr = (sc[28], sc[29])
    semg = (sc[30], sc[31])
    semw = (sc[32], sc[33])
    wid = lax.axis_index("s") * NC + lax.axis_index("c")
    base_pt = wid * PW
    iota = lax.iota(jnp.int32, 16)
    ccs = [jnp.full((16,), c, jnp.int32) for c in range(8)]

    def chunk_addr(cb):
        d = cb >> 14
        hw = cb & (S * S - 1)
        pbase = pl.multiple_of(d * 3 * (S * S) + hw, 8)
        return d, hw, pbase

    def read_descs(pbase, st, sem):
        return [pltpu.make_async_copy(coords_hbm.at[pl.ds(pbase, CW)],
                                      st[0], sem),
                pltpu.make_async_copy(coords_hbm.at[pl.ds(pbase + S * S, CW)],
                                      st[1], sem),
                pltpu.make_async_copy(
                    coords_hbm.at[pl.ds(pbase + 2 * S * S, CW)], st[2], sem)]

    def fire_reads(pbase, st, sem):
        cps = read_descs(pbase, st, sem)
        for cp in cps:
            cp.start()
        return cps

    def phase_a(st, d, hw):
        (cx_v, cy_v, cz_v, idx_v, msel_v, tx_v, ty_v, tz_v,
         rows_v, outd_v, outm_v, dx_v, dy_v, dz_v) = st
        d_f = d.astype(jnp.float32)

        def group_a(g):
            sl = pl.ds(g * 16, 16)
            cx = cx_v[sl]
            cy = cy_v[sl]
            cz = cz_v[sl]
            x0, tx = _unnormalize(cx)
            y0, ty = _unnormalize(cy)
            z0, tz = _unnormalize(cz)
            idx_v[sl] = (z0 << 14) + (y0 << 7) + x0
            msel_v[sl] = ((_round_bit(z0, tz) << 2)
                          + (_round_bit(y0, ty) << 1) + _round_bit(x0, tx))
            tx_v[sl] = tx
            ty_v[sl] = ty
            tz_v[sl] = tz
            hwl = hw + g * 16 + iota
            ph = (hwl >> 7).astype(jnp.float32)
            pw = (hwl & (S - 1)).astype(jnp.float32)
            dx_v[sl] = cx - d_f
            dy_v[sl] = cy - ph
            dz_v[sl] = cz - pw

        plsc.parallel_loop(0, GW, unroll=4)(group_a)

    def phase_b(st):
        (cx_v, cy_v, cz_v, idx_v, msel_v, tx_v, ty_v, tz_v,
         rows_v, outd_v, outm_v, dx_v, dy_v, dz_v) = st

        def group_b(g):
            pid = iota + g * 16
            v = [plsc.load_gather(rows_v, [pid, ccs[c]]) for c in range(8)]
            sl = pl.ds(g * 16, 16)
            tx = tx_v[sl]
            ty = ty_v[sl]
            tz = tz_v[sl]
            l00 = v[0] + tx * (v[1] - v[0])
            l01 = v[2] + tx * (v[3] - v[2])
            l10 = v[4] + tx * (v[5] - v[4])
            l11 = v[6] + tx * (v[7] - v[6])
            m0 = l00 + ty * (l01 - l00)
            m1 = l10 + ty * (l11 - l10)
            outd_v[sl] = m0 + tz * (m1 - m0)
            mw = plsc.load_gather(rows_v, [pid, msel_v[sl]])
            outm_v[sl] = (plsc.bitcast(mw, jnp.int32) & 1).astype(jnp.float32)

        plsc.parallel_loop(0, GW, unroll=4)(group_b)

    def fire_writes(cb, pbase, st, sem):
        return [pltpu.async_copy(st[9], outd_hbm.at[pl.ds(cb, CW)], sem),
                pltpu.async_copy(st[10], outm_hbm.at[pl.ds(cb, CW)], sem),
                pltpu.async_copy(st[11], disp_hbm.at[pl.ds(pbase, CW)], sem),
                pltpu.async_copy(st[12],
                                 disp_hbm.at[pl.ds(pbase + S * S, CW)], sem),
                pltpu.async_copy(st[13],
                                 disp_hbm.at[pl.ds(pbase + 2 * S * S, CW)],
                                 sem)]

    def pair_body(ci2, carry):
        cb0 = base_pt + (2 * ci2) * CW
        cb1 = cb0 + CW
        d0, hw0, pb0 = chunk_addr(cb0)
        d1, hw1, pb1 = chunk_addr(cb1)
        for cp in read_descs(pb0, sets[0], semr[0]):
            cp.wait()
        phase_a(sets[0], d0, hw0)
        g0 = pltpu.async_copy(tab_hbm.at[sets[0][3]], sets[0][8], semg[0])

        @pl.when(ci2 < NPAIR_W - 1)
        def _():
            _, _, pbn = chunk_addr(cb0 + 2 * CW)
            fire_reads(pbn, sets[0], semr[0])

        for cp in read_descs(pb1, sets[1], semr[1]):
            cp.wait()
        phase_a(sets[1], d1, hw1)
        g1 = pltpu.async_copy(tab_hbm.at[sets[1][3]], sets[1][8], semg[1])

        @pl.when(ci2 < NPAIR_W - 1)
        def _():
            _, _, pbn = chunk_addr(cb1 + 2 * CW)
            fire_reads(pbn, sets[1], semr[1])

        g0.wait()
        phase_b(sets[0])
        w0 = fire_writes(cb0, pb0, sets[0], semw[0])
        g1.wait()
        phase_b(sets[1])
        w1 = fire_writes(cb1, pb1, sets[1], semw[1])
        for cp in w0 + w1:
            cp.wait()
        return carry

    _, _, pbp0 = chunk_addr(base_pt)
    _, _, pbp1 = chunk_addr(base_pt + CW)
    fire_reads(pbp0, sets[0], semr[0])
    fire_reads(pbp1, sets[1], semr[1])
    lax.fori_loop(0, NPAIR_W, pair_body, 0)


def _warp_set_types():
    return [
        pltpu.VMEM((CW,), jnp.float32),
        pltpu.VMEM((CW,), jnp.float32),
        pltpu.VMEM((CW,), jnp.float32),
        pltpu.VMEM((CW,), jnp.int32),
        pltpu.VMEM((CW,), jnp.int32),
        pltpu.VMEM((CW,), jnp.float32),
        pltpu.VMEM((CW,), jnp.float32),
        pltpu.VMEM((CW,), jnp.float32),
        pltpu.VMEM((CW, 8), jnp.float32),
        pltpu.VMEM((CW,), jnp.float32),
        pltpu.VMEM((CW,), jnp.float32),
        pltpu.VMEM((CW,), jnp.float32),
        pltpu.VMEM((CW,), jnp.float32),
        pltpu.VMEM((CW,), jnp.float32),
    ]


@functools.cache
def _get_build():
    bufset = [
        pltpu.VMEM((RD2,), jnp.float32),
        pltpu.VMEM((RD2,), jnp.float32),
        pltpu.VMEM((RD2,), jnp.float32),
        pltpu.VMEM((RD2,), jnp.float32),
        pltpu.VMEM((CB, 8), jnp.float32),
    ]
    return functools.partial(
        pl.kernel,
        out_type=jax.ShapeDtypeStruct((V, 8), jnp.float32),
        mesh=_mesh(),
        compiler_params=_params,
        scratch_types=(bufset + bufset
                       + [pltpu.SemaphoreType.DMA] * 4),
    )(_build_body)


@functools.cache
def _get_warp():
    return functools.partial(
        pl.kernel,
        out_type=(
            jax.ShapeDtypeStruct((V,), jnp.float32),
            jax.ShapeDtypeStruct((V,), jnp.float32),
            jax.ShapeDtypeStruct((V * 3,), jnp.float32),
        ),
        mesh=_mesh(),
        compiler_params=_params,
        scratch_types=(_warp_set_types() + _warp_set_types()
                       + [pltpu.SemaphoreType.DMA] * 6),
    )(_warp_body)


def kernel(density, mask, warped_coords):
    den = density.reshape(V)
    msk = mask.reshape(V)
    coords = warped_coords.transpose(0, 1, 4, 2, 3).reshape(V * 3)
    tab = _get_build()(den, msk)
    outd, outm, disp = _get_warp()(coords, tab)
    disp = disp.reshape(S, 3, S, S).transpose(0, 2, 3, 1)
    return (outd.reshape(S, S, S), outm.reshape(S, S, S), disp)
